# baseline (speedup 1.0000x reference)
import jax
import jax.numpy as jnp
from jax.experimental import pallas as pl
from jax.experimental.pallas import tpu as pltpu

_BLOCK = 25000
_PE_PAD = 32


def _fused_kernel(x_ref, ids_ref, w_ref, b_ref, pe_ref, out_ref):
    x_blk = x_ref[...]
    d_out = w_ref.shape[1]
    h = jnp.dot(x_blk, w_ref[...], preferred_element_type=jnp.float32)
    out_ref[:, :d_out] = h + b_ref[0, :]
    ids = ids_ref[0, :, :]
    onehot_t = (jax.lax.broadcasted_iota(jnp.int32, (_PE_PAD, x_blk.shape[0]), 0)
                == ids).astype(jnp.float32)
    pos = jax.lax.dot_general(
        onehot_t, pe_ref[...],
        dimension_numbers=(((0,), (0,)), ((), ())),
        preferred_element_type=jnp.float32)
    out_ref[:, d_out:] = pos


def kernel(x, batch, W, b, pe_table):
    n, dim_in = x.shape
    d_out = W.shape[1]
    dim_pe = pe_table.shape[1]
    nb = -(-n // _BLOCK)
    ids_padded = jnp.zeros((nb * _BLOCK,), jnp.int32).at[:n].set(batch.astype(jnp.int32))
    ids3 = ids_padded.reshape(nb, 1, _BLOCK)
    b2 = b.reshape(1, d_out)
    pe_pad = jnp.zeros((_PE_PAD, dim_pe), jnp.float32).at[:pe_table.shape[0]].set(pe_table)

    return pl.pallas_call(
        _fused_kernel,
        grid=(nb,),
        in_specs=[
            pl.BlockSpec((_BLOCK, dim_in), lambda i: (i, 0)),
            pl.BlockSpec((1, 1, _BLOCK), lambda i: (i, 0, 0)),
            pl.BlockSpec((dim_in, d_out), lambda i: (0, 0)),
            pl.BlockSpec((1, d_out), lambda i: (0, 0)),
            pl.BlockSpec((_PE_PAD, dim_pe), lambda i: (0, 0)),
        ],
        out_specs=pl.BlockSpec((_BLOCK, d_out + dim_pe), lambda i: (i, 0)),
        out_shape=jax.ShapeDtypeStruct((n, d_out + dim_pe), jnp.float32),
        compiler_params=pltpu.CompilerParams(
            dimension_semantics=("arbitrary",),
        ),
    )(x, ids3, W, b2, pe_pad)


# speedup vs baseline: 1.0888x; 1.0888x over previous
import jax
import jax.numpy as jnp
from jax.experimental import pallas as pl
from jax.experimental.pallas import tpu as pltpu

_BLOCK = 14288
_PE_PAD = 32


def _fused_kernel(x_ref, ids_ref, w_ref, b_ref, pe_ref, out_ref):
    x_blk = x_ref[...]
    d_out = w_ref.shape[1]
    h = jnp.dot(x_blk, w_ref[...], preferred_element_type=jnp.float32)
    out_ref[:, :d_out] = h + b_ref[0, :]
    ids = ids_ref[0, :, :]
    onehot_t = (jax.lax.broadcasted_iota(jnp.int32, (_PE_PAD, x_blk.shape[0]), 0)
                == ids).astype(jnp.float32)
    pos = jax.lax.dot_general(
        onehot_t, pe_ref[...],
        dimension_numbers=(((0,), (0,)), ((), ())),
        preferred_element_type=jnp.float32)
    out_ref[:, d_out:] = pos


def kernel(x, batch, W, b, pe_table):
    n, dim_in = x.shape
    d_out = W.shape[1]
    dim_pe = pe_table.shape[1]
    nb = -(-n // _BLOCK)
    ids_padded = jnp.zeros((nb * _BLOCK,), jnp.int32).at[:n].set(batch.astype(jnp.int32))
    ids3 = ids_padded.reshape(nb, 1, _BLOCK)
    b2 = b.reshape(1, d_out)
    pe_pad = jnp.zeros((_PE_PAD, dim_pe), jnp.float32).at[:pe_table.shape[0]].set(pe_table)

    return pl.pallas_call(
        _fused_kernel,
        grid=(nb,),
        in_specs=[
            pl.BlockSpec((_BLOCK, dim_in), lambda i: (i, 0)),
            pl.BlockSpec((1, 1, _BLOCK), lambda i: (i, 0, 0)),
            pl.BlockSpec((dim_in, d_out), lambda i: (0, 0)),
            pl.BlockSpec((1, d_out), lambda i: (0, 0)),
            pl.BlockSpec((_PE_PAD, dim_pe), lambda i: (0, 0)),
        ],
        out_specs=pl.BlockSpec((_BLOCK, d_out + dim_pe), lambda i: (i, 0)),
        out_shape=jax.ShapeDtypeStruct((n, d_out + dim_pe), jnp.float32),
        compiler_params=pltpu.CompilerParams(
            dimension_semantics=("arbitrary",),
        ),
    )(x, ids3, W, b2, pe_pad)


# speedup vs baseline: 1.0925x; 1.0034x over previous
import jax
import jax.numpy as jnp
from jax.experimental import pallas as pl
from jax.experimental.pallas import tpu as pltpu

_BLOCK = 14288
_PE_PAD = 32


def _fused_kernel(x_ref, ids_ref, w_ref, b_ref, pe_ref, out_ref):
    x_blk = x_ref[...]
    d_out = w_ref.shape[1]
    h = jnp.dot(x_blk, w_ref[...], preferred_element_type=jnp.float32)
    out_ref[:, :d_out] = h + b_ref[0, :]
    ids = ids_ref[0, :, :]
    onehot_t = (jax.lax.broadcasted_iota(jnp.int32, (_PE_PAD, x_blk.shape[0]), 0)
                == ids).astype(jnp.float32)
    pos = jax.lax.dot_general(
        onehot_t, pe_ref[...],
        dimension_numbers=(((0,), (0,)), ((), ())),
        preferred_element_type=jnp.float32)
    out_ref[:, d_out:] = pos


def kernel(x, batch, W, b, pe_table):
    n, dim_in = x.shape
    d_out = W.shape[1]
    dim_pe = pe_table.shape[1]
    nb = -(-n // _BLOCK)
    ids_padded = jnp.zeros((nb * _BLOCK,), jnp.int32).at[:n].set(batch.astype(jnp.int32))
    ids3 = ids_padded.reshape(nb, 1, _BLOCK)
    b2 = b.reshape(1, d_out)
    pe_pad = jnp.zeros((_PE_PAD, dim_pe), jnp.float32).at[:pe_table.shape[0]].set(pe_table)

    return pl.pallas_call(
        _fused_kernel,
        grid=(nb,),
        in_specs=[
            pl.BlockSpec((_BLOCK, dim_in), lambda i: (i, 0)),
            pl.BlockSpec((1, 1, _BLOCK), lambda i: (i, 0, 0)),
            pl.BlockSpec((dim_in, d_out), lambda i: (0, 0)),
            pl.BlockSpec((1, d_out), lambda i: (0, 0)),
            pl.BlockSpec((_PE_PAD, dim_pe), lambda i: (0, 0)),
        ],
        out_specs=pl.BlockSpec((_BLOCK, d_out + dim_pe), lambda i: (i, 0)),
        out_shape=jax.ShapeDtypeStruct((n, d_out + dim_pe), jnp.float32),
        compiler_params=pltpu.CompilerParams(
            dimension_semantics=("parallel",),
        ),
    )(x, ids3, W, b2, pe_pad)
